# baseline (speedup 1.0000x reference)
import functools

import jax
import jax.numpy as jnp
from jax.experimental import pallas as pl
from jax.experimental.pallas import tpu as pltpu

_CONTRACT_LAST = (((1,), (1,)), ((), ()))


def _fused_kernel(d_ref, w1_ref, b1_ref, w2_ref, b2_ref, w3_ref, b3_ref,
                  x_ref, o_ref, g_ref, bcol_ref, *, B, C, RC):
    i = pl.program_id(0)

    @pl.when(i == 0)
    def _():
        h = jax.lax.dot_general(w1_ref[...], d_ref[...], _CONTRACT_LAST,
                                preferred_element_type=jnp.float32)
        h = jnp.maximum(h + b1_ref[...], 0.0)
        h = jnp.dot(w2_ref[...], h, preferred_element_type=jnp.float32)
        h = jnp.maximum(h + b2_ref[...], 0.0)
        for b in range(B):
            col = jnp.dot(w3_ref[...], h[:, b:b + 1],
                          preferred_element_type=jnp.float32) + b3_ref[...]
            g_ref[b * C:(b + 1) * C, :] = col[0:C, :] + 1.0
            bcol_ref[b * C:(b + 1) * C, :] = col[C:2 * C, :]

    g = g_ref[pl.ds(i * RC, RC), :]
    bb = bcol_ref[pl.ds(i * RC, RC), :]
    o_ref[...] = g * x_ref[...] + bb


def kernel(x, d, w1, b1, w2, b2, w3, b3):
    B, C, H, W = x.shape
    HW = H * W
    x_flat = x.reshape(B * C, HW)
    b1r = b1.reshape(-1, 1)
    b2r = b2.reshape(-1, 1)
    b3r = b3.reshape(-1, 1)

    R = 4
    RC = R * C

    def whole(shape):
        n = len(shape)
        return pl.BlockSpec(shape, lambda i, _n=n: (0,) * _n)

    out = pl.pallas_call(
        functools.partial(_fused_kernel, B=B, C=C, RC=RC),
        out_shape=jax.ShapeDtypeStruct((B * C, HW), jnp.float32),
        grid=(B // R,),
        in_specs=[whole(d.shape), whole(w1.shape), whole(b1r.shape),
                  whole(w2.shape), whole(b2r.shape),
                  whole(w3.shape), whole(b3r.shape),
                  pl.BlockSpec((RC, HW), lambda i: (i, 0))],
        out_specs=pl.BlockSpec((RC, HW), lambda i: (i, 0)),
        scratch_shapes=[pltpu.VMEM((B * C, 1), jnp.float32),
                        pltpu.VMEM((B * C, 1), jnp.float32)],
        compiler_params=pltpu.CompilerParams(
            dimension_semantics=("arbitrary",),
            vmem_limit_bytes=44 << 20,
            skip_device_barrier=True),
    )(d, w1, b1r, w2, b2r, w3, b3r, x_flat)

    return out.reshape(B, C, H, W)


# speedup vs baseline: 1.0346x; 1.0346x over previous
import functools

import jax
import jax.numpy as jnp
from jax.experimental import pallas as pl
from jax.experimental.pallas import tpu as pltpu

_CONTRACT_LAST = (((1,), (1,)), ((), ()))


def _fused_kernel(d_ref, w1_ref, b1_ref, w2_ref, b2_ref, w3_ref, b3_ref,
                  x_ref, o_ref, g_ref, bcol_ref, *, B, C, RC):
    i = pl.program_id(0)

    @pl.when(i == 0)
    def _():
        h = jax.lax.dot_general(w1_ref[...], d_ref[...], _CONTRACT_LAST,
                                preferred_element_type=jnp.float32)
        h = jnp.maximum(h + b1_ref[...], 0.0)
        h = jnp.dot(w2_ref[...], h, preferred_element_type=jnp.float32)
        h = jnp.maximum(h + b2_ref[...], 0.0)
        for b in range(B):
            col = jnp.dot(w3_ref[...], h[:, b:b + 1],
                          preferred_element_type=jnp.float32) + b3_ref[...]
            g_ref[b * C:(b + 1) * C, :] = col[0:C, :] + 1.0
            bcol_ref[b * C:(b + 1) * C, :] = col[C:2 * C, :]

    g = g_ref[pl.ds(i * RC, RC), :]
    bb = bcol_ref[pl.ds(i * RC, RC), :]
    o_ref[...] = g * x_ref[...] + bb


def kernel(x, d, w1, b1, w2, b2, w3, b3):
    B, C, H, W = x.shape
    HW = H * W
    x_flat = x.reshape(B * C, HW)
    b1r = b1.reshape(-1, 1)
    b2r = b2.reshape(-1, 1)
    b3r = b3.reshape(-1, 1)

    R = 8
    RC = R * C

    def whole(shape):
        n = len(shape)
        return pl.BlockSpec(shape, lambda i, _n=n: (0,) * _n)

    out = pl.pallas_call(
        functools.partial(_fused_kernel, B=B, C=C, RC=RC),
        out_shape=jax.ShapeDtypeStruct((B * C, HW), jnp.float32),
        grid=(B // R,),
        in_specs=[whole(d.shape), whole(w1.shape), whole(b1r.shape),
                  whole(w2.shape), whole(b2r.shape),
                  whole(w3.shape), whole(b3r.shape),
                  pl.BlockSpec((RC, HW), lambda i: (i, 0))],
        out_specs=pl.BlockSpec((RC, HW), lambda i: (i, 0)),
        scratch_shapes=[pltpu.VMEM((B * C, 1), jnp.float32),
                        pltpu.VMEM((B * C, 1), jnp.float32)],
        compiler_params=pltpu.CompilerParams(
            dimension_semantics=("arbitrary",),
            vmem_limit_bytes=44 << 20),
    )(d, w1, b1r, w2, b2r, w3, b3r, x_flat)

    return out.reshape(B, C, H, W)
